# baseline (speedup 1.0000x reference)
import functools
import jax
import jax.numpy as jnp
from jax import lax
from jax.experimental import pallas as pl
from jax.experimental.pallas import tpu as pltpu
from jax.experimental.pallas import tpu_sc as plsc

N_FIELDS = 26
VOCAB = 100000
D = 32
H = 64
CH = 416


def _make_sc_gather(n_rows):
    info = plsc.get_sparse_core_info()
    nc, ns = info.num_cores, info.num_subcores
    nw = nc * ns
    rows_per_w = n_rows // nw
    n_chunks = rows_per_w // CH
    mesh = plsc.VectorSubcoreMesh(core_axis_name="c", subcore_axis_name="s")

    @functools.partial(
        pl.kernel,
        mesh=mesh,
        compiler_params=pltpu.CompilerParams(use_tc_tiling_on_sc=True),
        out_type=jax.ShapeDtypeStruct((n_rows, 4 * D), jnp.float32),
        scratch_types=[
            pltpu.VMEM((rows_per_w,), jnp.int32),
            pltpu.VMEM((CH, 4 * D), jnp.float32),
            pltpu.SemaphoreType.DMA,
        ],
    )
    def gather_k(table_hbm, idx128_hbm, out_hbm, idx_v, rows_v, sem):
        wid = lax.axis_index("s") * nc + lax.axis_index("c")
        base = wid * rows_per_w
        pltpu.sync_copy(idx128_hbm.at[pl.ds(base, rows_per_w)], idx_v)

        def chunk(t, carry):
            co = t * CH
            pltpu.async_copy(table_hbm.at[idx_v.at[pl.ds(co, CH)]],
                             rows_v, sem).wait()
            pltpu.sync_copy(rows_v, out_hbm.at[pl.ds(base + co, CH)])
            return carry

        lax.fori_loop(0, n_chunks, chunk, 0, unroll=False)

    return gather_k


def _mlp_body(g_ref, x_ref, w1_ref, b1_ref, w2_ref, b2_ref, o_ref):
    blk = x_ref.shape[0]
    lane_chunk = jax.lax.broadcasted_iota(jnp.int32, (blk, 4 * D), 1) >> 5
    acc = None
    for f in range(N_FIELDS):
        c = (x_ref[:, f:f + 1] & 3)
        gm = jnp.where(lane_chunk == c, g_ref[f], 0.0)
        p = jnp.dot(gm, w1_ref[f], preferred_element_type=jnp.float32)
        acc = p if acc is None else acc + p
    h = jnp.maximum(acc + b1_ref[...], 0.0)
    o_ref[...] = jnp.sum(h * w2_ref[...], axis=1, keepdims=True) + b2_ref[...]


def kernel(x, tables, W1, b1, W2, b2):
    batch = x.shape[0]
    n_rows = batch * N_FIELDS
    table128 = tables.reshape(N_FIELDS * VOCAB // 4, 4 * D)
    offs = (jnp.arange(N_FIELDS, dtype=jnp.int32) * VOCAB)[:, None]
    idx128 = (x.astype(jnp.int32).T + offs).reshape(n_rows) >> 2

    g = _make_sc_gather(n_rows)(table128, idx128)
    g3 = g.reshape(N_FIELDS, batch, 4 * D)
    w1t = jnp.tile(W1.reshape(N_FIELDS, D, H), (1, 4, 1))

    blk = 512
    out = pl.pallas_call(
        _mlp_body,
        grid=(batch // blk,),
        in_specs=[
            pl.BlockSpec((N_FIELDS, blk, 4 * D), lambda i: (0, i, 0)),
            pl.BlockSpec((blk, N_FIELDS), lambda i: (i, 0)),
            pl.BlockSpec((N_FIELDS, 4 * D, H), lambda i: (0, 0, 0)),
            pl.BlockSpec((1, H), lambda i: (0, 0)),
            pl.BlockSpec((1, H), lambda i: (0, 0)),
            pl.BlockSpec((1, 1), lambda i: (0, 0)),
        ],
        out_specs=pl.BlockSpec((blk, 1), lambda i: (i, 0)),
        out_shape=jax.ShapeDtypeStruct((batch, 1), jnp.float32),
    )(g3, x.astype(jnp.int32), w1t, b1.reshape(1, H), W2.reshape(1, H),
      b2.reshape(1, 1))
    return out


# speedup vs baseline: 1.7791x; 1.7791x over previous
import functools
import jax
import jax.numpy as jnp
from jax import lax
from jax.experimental import pallas as pl
from jax.experimental.pallas import tpu as pltpu
from jax.experimental.pallas import tpu_sc as plsc

N_FIELDS = 26
VOCAB = 100000
D = 32
H = 64
FD = N_FIELDS * D


def _make_sc_gather(batch):
    info = plsc.get_sparse_core_info()
    nc, ns = info.num_cores, info.num_subcores
    nw = nc * ns
    rows_per_w = FD // nw
    mesh = plsc.VectorSubcoreMesh(core_axis_name="c", subcore_axis_name="s")

    @functools.partial(
        pl.kernel,
        mesh=mesh,
        compiler_params=pltpu.CompilerParams(use_tc_tiling_on_sc=False),
        out_type=jax.ShapeDtypeStruct((FD, batch), jnp.float32),
        scratch_types=[
            pltpu.VMEM((batch,), jnp.int32),
            pltpu.VMEM((batch,), jnp.int32),
            pltpu.VMEM((batch,), jnp.float32),
            pltpu.SemaphoreType.DMA,
        ],
    )
    def gather_k(table_hbm, xt_hbm, out_hbm, xf_v, idx_v, row_v, sem):
        wid = lax.axis_index("s") * nc + lax.axis_index("c")
        p0 = wid * rows_per_w
        lane = lax.iota(jnp.int32, 16)

        def pair(i, carry):
            p = p0 + i
            f = p // D
            base = p * VOCAB
            pltpu.sync_copy(xt_hbm.at[f], xf_v)

            def addv(j, carry2):
                idx_v[pl.ds(j * 16, 16)] = xf_v[pl.ds(j * 16, 16)] + base
                return carry2

            lax.fori_loop(0, batch // 16, addv, 0, unroll=False)
            pltpu.async_copy(table_hbm.at[idx_v], row_v, sem).wait()
            pltpu.sync_copy(row_v, out_hbm.at[p])
            return carry

        lax.fori_loop(0, rows_per_w, pair, 0, unroll=False)

    return gather_k


def _mlp_body(e_ref, w1t_ref, b1_ref, w2_ref, b2_ref, o_ref):
    ht = jnp.dot(w1t_ref[...], e_ref[...],
                 preferred_element_type=jnp.float32)
    ht = jnp.maximum(ht + b1_ref[...], 0.0)
    o_ref[...] = jnp.dot(w2_ref[...], ht,
                         preferred_element_type=jnp.float32) + b2_ref[...]


def kernel(x, tables, W1, b1, W2, b2):
    batch = x.shape[0]
    tlin = jnp.transpose(tables, (0, 2, 1)).reshape(N_FIELDS * D * VOCAB)
    xt = jnp.transpose(x.astype(jnp.int32), (1, 0))

    e3 = _make_sc_gather(batch)(tlin, xt)

    blk = 1024
    w1t = jnp.transpose(W1, (1, 0))
    outT = pl.pallas_call(
        _mlp_body,
        grid=(batch // blk,),
        in_specs=[
            pl.BlockSpec((FD, blk), lambda i: (0, i)),
            pl.BlockSpec((H, FD), lambda i: (0, 0)),
            pl.BlockSpec((H, 1), lambda i: (0, 0)),
            pl.BlockSpec((1, H), lambda i: (0, 0)),
            pl.BlockSpec((1, 1), lambda i: (0, 0)),
        ],
        out_specs=pl.BlockSpec((1, blk), lambda i: (0, i)),
        out_shape=jax.ShapeDtypeStruct((1, batch), jnp.float32),
    )(e3, w1t, b1.reshape(H, 1), W2.reshape(1, H), b2.reshape(1, 1))
    return outT.reshape(batch, 1)
